# Initial kernel scaffold; baseline (speedup 1.0000x reference)
import jax
import jax.numpy as jnp
from jax.experimental import pallas as pl


def kernel(x, batch, node_attr, edge_src, edge_dst, emb, Wpu, W1r, W2r, L1, Ae, Na, S, h, mix):
    raise NotImplementedError("write your pallas kernel here")


# speedup vs baseline: 1.9343x; 1.9343x over previous
import functools
import math

import numpy as np
import jax
import jax.numpy as jnp
from jax import lax
from jax.experimental import pallas as pl
from jax.experimental.pallas import tpu as pltpu
from jax.experimental.pallas import tpu_sc as plsc

N = 50000
E = 800000
DH = 32
NS = 8
NG = 8
DG = 24
DZ = 40
NB = 8
ED = 16
MAXR = 5.0
LAYERS = 4
NATOM = 100
RH = 16

TW = 48
XPW = 16

BN = 2000
NBN = N // BN

CH = 1024
SUB = 128
NSUB = 8
NCHUNK = 800
EP = NCHUNK * CH
EPAD = EP - E
NTILES = 32
CPT = NCHUNK // NTILES
CPS = NCHUNK // 16
NHALF = N // 2

BE = 4096
NBE = EP // BE

SQRT3 = math.sqrt(3.0)
BESC = math.sqrt(2.0 / MAXR) * math.sqrt(float(NB))
PI = math.pi
TWO_PI = 2.0 * math.pi
INV_TWO_PI = 1.0 / TWO_PI


def _sin_poly(t):
    q = jnp.floor(t * INV_TWO_PI + 0.5)
    r = t - q * TWO_PI
    r = jnp.where(r > 0.5 * PI, PI - r, r)
    r = jnp.where(r < -0.5 * PI, -PI - r, r)
    x2 = r * r
    s = -1.0 / 39916800.0
    s = s * x2 + 1.0 / 362880.0
    s = s * x2 - 1.0 / 5040.0
    s = s * x2 + 1.0 / 120.0
    s = s * x2 - 1.0 / 6.0
    s = s * x2 + 1.0
    return r * s


def _edge_feats(ev):
    n2 = jnp.sum(ev * ev, axis=1, keepdims=True)
    el = jnp.sqrt(n2)
    denom = jnp.where(el == 0.0, 1e-9, el)
    sh = SQRT3 * ev / denom
    safe = jnp.where(el == 0.0, 1e-9, el)
    th = (PI / MAXR) * safe
    s1 = _sin_poly(th)
    c1 = _sin_poly(th + 0.5 * PI)
    sks = [s1, 2.0 * c1 * s1]
    for _ in range(NB - 2):
        sks.append(2.0 * c1 * sks[-1] - sks[-2])
    ef = jnp.concatenate(sks, axis=1) * (BESC / safe)
    cut = jnp.where(el > MAXR, 0.0, jnp.where(el < 0.5 * MAXR, 1.0, s1 * s1))
    ea = cut * sh
    return ea, ef


def _k0_body(x_ref, attr_ref, qt_ref, emb_ref, nac_ref, t_ref, xp_ref, cn_ref):
    x = x_ref[...]
    y = jnp.dot(x, qt_ref[...], preferred_element_type=jnp.float32)
    pad10 = jnp.zeros((BN, 10), jnp.float32)
    t_ref[...] = jnp.concatenate([y, x, pad10], axis=1)
    xp_ref[...] = jnp.concatenate([x, pad10], axis=1)
    attr = attr_ref[0, 0, :]
    oh = (attr[:, None] == lax.broadcasted_iota(jnp.int32, (BN, NATOM), 1))
    oh = oh.astype(jnp.float32)
    en = jnp.dot(emb_ref[...], nac_ref[...], preferred_element_type=jnp.float32)
    for l in range(LAYERS):
        cn_ref[l] = jnp.dot(oh, en[:, l * DZ:(l + 1) * DZ],
                            preferred_element_type=jnp.float32)


def _node_init(x, attr3, QT, emb, NaC):
    return pl.pallas_call(
        _k0_body,
        grid=(NBN,),
        in_specs=[
            pl.BlockSpec((BN, 6), lambda b: (b, 0)),
            pl.BlockSpec((1, 1, BN), lambda b: (b, 0, 0)),
            pl.BlockSpec((6, DH), lambda b: (0, 0)),
            pl.BlockSpec((NATOM, ED), lambda b: (0, 0)),
            pl.BlockSpec((ED, 4 * DZ), lambda b: (0, 0)),
        ],
        out_specs=[
            pl.BlockSpec((BN, TW), lambda b: (b, 0)),
            pl.BlockSpec((BN, XPW), lambda b: (b, 0)),
            pl.BlockSpec((LAYERS, BN, DZ), lambda b: (0, b, 0)),
        ],
        out_shape=[
            jax.ShapeDtypeStruct((N, TW), jnp.float32),
            jax.ShapeDtypeStruct((N, XPW), jnp.float32),
            jax.ShapeDtypeStruct((LAYERS, N, DZ), jnp.float32),
        ],
    )(x, attr3, QT, emb, NaC)


def _sc_gather(T, XP, srcR, dstR):
    mesh = plsc.VectorSubcoreMesh(core_axis_name="c", subcore_axis_name="s")

    @functools.partial(
        pl.kernel,
        mesh=mesh,
        out_type=(
            jax.ShapeDtypeStruct((EP, TW), jnp.float32),
            jax.ShapeDtypeStruct((EP, XPW), jnp.float32),
        ),
        scratch_types=[
            pltpu.VMEM((NSUB, SUB), jnp.int32),
            pltpu.VMEM((NSUB, SUB), jnp.int32),
            pltpu.VMEM((CH, TW), jnp.float32),
            pltpu.VMEM((CH, XPW), jnp.float32),
            pltpu.SemaphoreType.DMA,
            pltpu.SemaphoreType.DMA,
        ],
        compiler_params=pltpu.CompilerParams(use_tc_tiling_on_sc=False),
    )
    def gather_k(t_hbm, xp_hbm, src_hbm, dst_hbm, g_hbm, xd_hbm,
                 isv, idv, tb, xb, semt, semx):
        wid = lax.axis_index("s") * 2 + lax.axis_index("c")

        def chunk(k, carry):
            cid = wid * CPT + k
            base = cid * CH
            pltpu.sync_copy(src_hbm.at[cid], isv)
            pltpu.sync_copy(dst_hbm.at[cid], idv)
            cps = [
                pltpu.async_copy(t_hbm.at[isv.at[j]],
                                 tb.at[pl.ds(j * SUB, SUB)], semt)
                for j in range(NSUB)
            ]
            cpx = [
                pltpu.async_copy(xp_hbm.at[idv.at[j]],
                                 xb.at[pl.ds(j * SUB, SUB)], semx)
                for j in range(NSUB)
            ]
            for cp in cps:
                cp.wait()
            for cp in cpx:
                cp.wait()
            pltpu.sync_copy(tb, g_hbm.at[pl.ds(base, CH)])
            pltpu.sync_copy(xb, xd_hbm.at[pl.ds(base, CH)])
            return carry

        lax.fori_loop(0, CPT, chunk, 0)

    return gather_k(T, XP, srcR, dstR)


def _k2_body(g_ref, xd_ref, w1_ref, w2_ref, l1_ref, ae_ref, out_ref):
    g = g_ref[...]
    xd = xd_ref[...]
    ysrc = g[:, 0:DH]
    ev_r = g[:, DH:DH + 3] - xd[:, 0:3]
    ev_v = g[:, DH + 3:DH + 6] - xd[:, 3:6]
    ea_r, ef_r = _edge_feats(ev_r)
    ea_v, ef_v = _edge_feats(ev_v)
    ea = jnp.concatenate([ea_r, ea_v], axis=1)
    ef = jnp.concatenate([ef_r, ef_v], axis=1)
    hmid = jnp.dot(ef, w1_ref[...], preferred_element_type=jnp.float32)
    hmid = hmid * (1.0 / (1.0 + jnp.exp(-hmid)))
    rw = jnp.dot(hmid, w2_ref[...], preferred_element_type=jnp.float32)
    lin = jnp.dot(ysrc, l1_ref[...], preferred_element_type=jnp.float32)
    lin = lin + jnp.dot(ea, ae_ref[...], preferred_element_type=jnp.float32)
    out_ref[...] = lin * rw


def _edge_stage(G, XD, W1i, W2i, L1i, Aei):
    return pl.pallas_call(
        _k2_body,
        grid=(NBE,),
        in_specs=[
            pl.BlockSpec((BE, TW), lambda b: (b, 0)),
            pl.BlockSpec((BE, XPW), lambda b: (b, 0)),
            pl.BlockSpec((2 * NB, RH), lambda b: (0, 0)),
            pl.BlockSpec((RH, DZ), lambda b: (0, 0)),
            pl.BlockSpec((DH, DZ), lambda b: (0, 0)),
            pl.BlockSpec((6, DZ), lambda b: (0, 0)),
        ],
        out_specs=pl.BlockSpec((BE, DZ), lambda b: (b, 0)),
        out_shape=jax.ShapeDtypeStruct((EP, DZ), jnp.float32),
    )(G, XD, W1i, W2i, L1i, Aei)


def _sc_scatter(MSG, dstR, zrow):
    mesh = plsc.VectorSubcoreMesh(core_axis_name="c", subcore_axis_name="s")

    @functools.partial(
        pl.kernel,
        mesh=mesh,
        out_type=jax.ShapeDtypeStruct((N, DZ), jnp.float32),
        scratch_types=[
            pltpu.VMEM((NSUB, SUB), jnp.int32),
            pltpu.VMEM((CH, DZ), jnp.float32),
            pltpu.VMEM_SHARED((NHALF + 16, DZ), jnp.float32),
        ],
        compiler_params=pltpu.CompilerParams(use_tc_tiling_on_sc=False),
    )
    def scatter_k(msg_hbm, dst_hbm, z_hbm, agg_hbm, idv, mb, acc):
        c = lax.axis_index("c")
        s = lax.axis_index("s")
        lo = c * NHALF
        pltpu.sync_copy(z_hbm, mb)
        for t in range(25):
            @pl.when(s == t % 16)
            def _():
                pltpu.sync_copy(mb.at[pl.ds(0, 1000)],
                                acc.at[pl.ds(t * 1000, 1000)])
        @pl.when(s == 15)
        def _():
            pltpu.sync_copy(mb.at[pl.ds(0, 16)], acc.at[pl.ds(NHALF, 16)])
        plsc.subcore_barrier()

        def chunk(k, carry):
            cid = s * CPS + k
            pltpu.sync_copy(dst_hbm.at[cid], idv)
            pltpu.sync_copy(msg_hbm.at[pl.ds(cid * CH, CH)], mb)
            for j in range(NSUB):
                for v in range(SUB // 16):
                    iv = idv[j, pl.ds(v * 16, 16)]
                    gidx = lax.iota(jnp.int32, 16) + (cid * CH + j * SUB + v * 16)
                    m = (iv >= lo) & (iv < lo + NHALF) & (gidx < E)
                    idv[j, pl.ds(v * 16, 16)] = jnp.where(m, iv - lo, NHALF)
            for j in range(NSUB):
                pltpu.sync_copy(mb.at[pl.ds(j * SUB, SUB)],
                                acc.at[idv.at[j]], add=True)
            return carry

        lax.fori_loop(0, CPS, chunk, 0)
        plsc.subcore_barrier()
        for t in range(25):
            @pl.when(s == t % 16)
            def _():
                pltpu.sync_copy(acc.at[pl.ds(t * 1000, 1000)],
                                agg_hbm.at[pl.ds(lo + t * 1000, 1000)])

    return scatter_k(MSG, dstR, zrow)


def _k4_body(agg_ref, cn_ref, tc_ref, tp_ref, s_ref, q_ref, hv_ref,
             rr_ref, tn_ref, xpn_ref, x_ref):
    tcur = tc_ref[...]
    y = tcur[:, 0:DH]
    y_old = tp_ref[...][:, 0:DH]
    z = agg_ref[...] * 0.25 + cn_ref[0]
    zs = z[:, 0:NS]
    scal = zs * (1.0 / (1.0 + jnp.exp(-zs)))
    gates = 1.0 / (1.0 + jnp.exp(-z[:, NS:NS + NG]))
    grep = jnp.dot(gates, rr_ref[...], preferred_element_type=jnp.float32)
    gated = z[:, NS + NG:DZ] * grep
    y_new = jnp.concatenate([scal, gated], axis=1)
    y_new2 = jnp.dot(y, s_ref[...], preferred_element_type=jnp.float32)
    hm = hv_ref[0, 0]
    hm2 = hv_ref[0, 1]
    y_nx = 2.0 * y - y_old + hm * y_new + hm2 * y_new2
    x_nx = jnp.dot(y_nx, q_ref[...], preferred_element_type=jnp.float32)
    pad10 = jnp.zeros((BN, 10), jnp.float32)
    tn_ref[...] = jnp.concatenate([y_nx, x_nx, pad10], axis=1)
    xpn_ref[...] = jnp.concatenate([x_nx, pad10], axis=1)
    x_ref[...] = x_nx


def _node_update(AGG, Cn, i, Tc, Tp, Si, Q, hv, Rrep):
    return pl.pallas_call(
        _k4_body,
        grid=(NBN,),
        in_specs=[
            pl.BlockSpec((BN, DZ), lambda b: (b, 0)),
            pl.BlockSpec((1, BN, DZ), lambda b, i=i: (i, b, 0)),
            pl.BlockSpec((BN, TW), lambda b: (b, 0)),
            pl.BlockSpec((BN, TW), lambda b: (b, 0)),
            pl.BlockSpec((DH, DH), lambda b: (0, 0)),
            pl.BlockSpec((DH, 6), lambda b: (0, 0)),
            pl.BlockSpec((1, 2), lambda b: (0, 0)),
            pl.BlockSpec((NG, DG), lambda b: (0, 0)),
        ],
        out_specs=[
            pl.BlockSpec((BN, TW), lambda b: (b, 0)),
            pl.BlockSpec((BN, XPW), lambda b: (b, 0)),
            pl.BlockSpec((BN, 6), lambda b: (b, 0)),
        ],
        out_shape=[
            jax.ShapeDtypeStruct((N, TW), jnp.float32),
            jax.ShapeDtypeStruct((N, XPW), jnp.float32),
            jax.ShapeDtypeStruct((N, 6), jnp.float32),
        ],
    )(AGG, Cn, Tc, Tp, Si, Q, hv, Rrep)


def kernel(x, batch, node_attr, edge_src, edge_dst, emb, Wpu, W1r, W2r,
           L1, Ae, Na, S, h, mix):
    Q, _ = jnp.linalg.qr(Wpu)
    QT = Q.T
    NaC = jnp.concatenate([Na[0], Na[1], Na[2], Na[3]], axis=1)
    pad = jnp.zeros((EPAD,), edge_src.dtype)
    srcR = jnp.concatenate([edge_src, pad]).reshape(NCHUNK, NSUB, SUB)
    dstR = jnp.concatenate([edge_dst, pad]).reshape(NCHUNK, NSUB, SUB)
    zrow = jnp.zeros((CH, DZ), jnp.float32)
    Rrep = jnp.asarray(np.repeat(np.eye(NG, dtype=np.float32), 3, axis=1))
    attr3 = node_attr.reshape(NBN, 1, BN)

    T, XP, Cn = _node_init(x, attr3, QT, emb, NaC)
    Tp = T
    x_out = None
    for i in range(LAYERS):
        G, XD = _sc_gather(T, XP, srcR, dstR)
        MSG = _edge_stage(G, XD, W1r[i], W2r[i], L1[i], Ae[i])
        AGG = _sc_scatter(MSG, dstR, zrow)
        hv = jnp.stack([h[i] * h[i] * mix[i],
                        h[i] * h[i] * (mix[i] - 1.0)]).reshape(1, 2)
        Tn, XPn, x_out = _node_update(AGG, Cn, i, T, Tp, S[i], Q, hv, Rrep)
        Tp = T
        T = Tn
        XP = XPn
    return x_out
